# baseline (speedup 1.0000x reference)
import functools

import jax
import jax.numpy as jnp
from jax import lax
from jax.experimental import pallas as pl
from jax.experimental.pallas import tpu as pltpu
from jax.experimental.pallas import tpu_sc as plsc

T = 2048
D = 1024
E = 8
H = 4 * D
P = 2 * T
CH = 256
RB = 256
NB = P // RB
MAXT = NB + E - 1
HBLK = 2048
NHB = H // HBLK

NC = 2
NS = 16
NW = NC * NS


def _router_body(lg_ref, pos_ref, wrow_ref, offs_ref):
    lane = lax.broadcasted_iota(jnp.int32, (T, 128), 1)
    neg = jnp.float32(-1e30)
    logits = jnp.where(lane < E, lg_ref[...], neg)
    m1 = jnp.max(logits, axis=1, keepdims=True)
    i1 = jnp.min(jnp.where(logits == m1, lane, 127), axis=1, keepdims=True)
    l2 = jnp.where(lane == i1, neg, logits)
    m2 = jnp.max(l2, axis=1, keepdims=True)
    i2 = jnp.min(jnp.where(l2 == m2, lane, 127), axis=1, keepdims=True)
    w1 = 1.0 / (1.0 + jnp.exp(m2 - m1))
    w2 = 1.0 - w1

    oh1 = (lane == i1).astype(jnp.float32)
    oh2 = (lane == i2).astype(jnp.float32)

    li = lax.broadcasted_iota(jnp.int32, (CH, CH), 0)
    lj = lax.broadcasted_iota(jnp.int32, (CH, CH), 1)
    ltri = (li > lj).astype(jnp.float32)

    off = jnp.zeros((1, 128), jnp.float32)
    ranks = []
    for c in range(2 * T // CH):
        src = oh1 if c < T // CH else oh2
        cc = c % (T // CH)
        chunk = src[cc * CH:(cc + 1) * CH, :]
        ranks.append(jnp.dot(ltri, chunk, precision=lax.Precision.HIGHEST,
                             preferred_element_type=jnp.float32) + off)
        off = off + jnp.sum(chunk, axis=0, keepdims=True)

    counts = off
    ui = lax.broadcasted_iota(jnp.int32, (128, 128), 0)
    uj = lax.broadcasted_iota(jnp.int32, (128, 128), 1)
    uppr = (ui < uj).astype(jnp.float32)
    c_hi = jnp.floor(counts * (1.0 / 64.0))
    c_lo = counts - 64.0 * c_hi
    offs = 64.0 * jnp.dot(c_hi, uppr, precision=lax.Precision.HIGHEST,
                          preferred_element_type=jnp.float32) \
        + jnp.dot(c_lo, uppr, precision=lax.Precision.HIGHEST,
                  preferred_element_type=jnp.float32)

    rank_all = jnp.concatenate(ranks, axis=0)
    oh_all = jnp.concatenate([oh1, oh2], axis=0)
    pos = jnp.sum(oh_all * (rank_all + offs), axis=1, keepdims=True)
    pos_ref[...] = pos.astype(jnp.int32)
    w_all = jnp.concatenate([w1, w2], axis=0)
    wrow_ref[...] = jnp.broadcast_to(w_all, (P, 128))
    offs_ref[...] = offs.astype(jnp.int32)


def _router(logits_p):
    return pl.pallas_call(
        _router_body,
        in_specs=[
            pl.BlockSpec((T, 128), lambda: (0, 0)),
        ],
        out_specs=[
            pl.BlockSpec((P, 1), lambda: (0, 0)),
            pl.BlockSpec((P, 128), lambda: (0, 0)),
            pl.BlockSpec((1, 128), lambda: (0, 0)),
        ],
        out_shape=[
            jax.ShapeDtypeStruct((P, 1), jnp.int32),
            jax.ShapeDtypeStruct((P, 128), jnp.float32),
            jax.ShapeDtypeStruct((1, 128), jnp.int32),
        ],
    )(logits_p)


def _dispatch_body(pos_hbm, wrow_hbm, x_hbm, xs_hbm, wsort_hbm,
                   posv, tokv, rowbuf, wbuf, sem):
    wid = lax.axis_index("s") * NC + lax.axis_index("c")
    npair = P // NW
    half = npair // 2
    for hp in range(2):
        base = wid * npair + hp * half
        pltpu.sync_copy(pos_hbm.at[pl.ds(base, half)], posv)
        for j in range(half // 16):
            t16 = (base + j * 16 + lax.iota(jnp.int32, 16)) & (T - 1)
            tokv[pl.ds(j * 16, 16)] = t16
        pltpu.async_copy(x_hbm.at[tokv], rowbuf, sem).wait()
        pltpu.sync_copy(wrow_hbm.at[pl.ds(base, half)], wbuf)
        pltpu.async_copy(rowbuf, xs_hbm.at[posv], sem).wait()
        pltpu.async_copy(wbuf, wsort_hbm.at[posv], sem).wait()


_dispatch = functools.partial(
    pl.kernel,
    out_type=[
        jax.ShapeDtypeStruct((P, D), jnp.float32),
        jax.ShapeDtypeStruct((P, 128), jnp.float32),
    ],
    mesh=plsc.VectorSubcoreMesh(core_axis_name="c", subcore_axis_name="s",
                                num_cores=NC, num_subcores=NS),
    scratch_types=[
        pltpu.VMEM((P // NW // 2,), jnp.int32),
        pltpu.VMEM((P // NW // 2,), jnp.int32),
        pltpu.VMEM((P // NW // 2, D), jnp.float32),
        pltpu.VMEM((P // NW // 2, 128), jnp.float32),
        pltpu.SemaphoreType.DMA,
    ],
)(_dispatch_body)


def _gmm_body(tg_ref, tb_ref, tv_ref, offs_ref,
              xs_ref, ws_ref, w1_ref, b1_ref, w2_ref, b2_ref, out_ref):
    h = pl.program_id(0)
    i = pl.program_id(1)

    @pl.when((h == 0) & (i == 0))
    def _init():
        out_ref[...] = jnp.zeros_like(out_ref)

    @pl.when(tv_ref[i] == 1)
    def _compute():
        g = tg_ref[i]
        b = tb_ref[i]
        r0 = b * RB
        riota = r0 + lax.broadcasted_iota(jnp.int32, (RB, 1), 0)
        active = (riota >= offs_ref[g]) & (riota < offs_ref[g + 1])

        hpre = jnp.dot(xs_ref[...].astype(jnp.bfloat16),
                       w1_ref[0].astype(jnp.bfloat16),
                       preferred_element_type=jnp.float32) + b1_ref[0]
        hact = 0.5 * hpre * (1.0 + lax.erf(hpre * 0.7071067811865476))
        acc = jnp.dot(hact.astype(jnp.bfloat16),
                      w2_ref[0].astype(jnp.bfloat16),
                      preferred_element_type=jnp.float32)

        wcol = jnp.where(active, ws_ref[:, :1], 0.0)
        contrib = wcol * (acc + jnp.where(h == 0, 1.0, 0.0) * b2_ref[0])
        out_ref[pl.ds(r0, RB), :] += contrib


def _gmm(tile_g, tile_b, tile_v, offs9, xs, wsort, W1, b1r, W2, b2r):
    grid_spec = pltpu.PrefetchScalarGridSpec(
        num_scalar_prefetch=4,
        grid=(NHB, MAXT),
        in_specs=[
            pl.BlockSpec((RB, D), lambda h, i, tg, tb, tv, of: (tb[i], 0)),
            pl.BlockSpec((RB, 128), lambda h, i, tg, tb, tv, of: (tb[i], 0)),
            pl.BlockSpec((1, D, HBLK),
                         lambda h, i, tg, tb, tv, of: (tg[i], 0, h)),
            pl.BlockSpec((1, 1, HBLK),
                         lambda h, i, tg, tb, tv, of: (tg[i], 0, h)),
            pl.BlockSpec((1, HBLK, D),
                         lambda h, i, tg, tb, tv, of: (tg[i], h, 0)),
            pl.BlockSpec((1, 1, D),
                         lambda h, i, tg, tb, tv, of: (tg[i], 0, 0)),
        ],
        out_specs=pl.BlockSpec((P, D), lambda h, i, tg, tb, tv, of: (0, 0)),
    )
    return pl.pallas_call(
        _gmm_body,
        grid_spec=grid_spec,
        out_shape=jax.ShapeDtypeStruct((P, D), jnp.float32),
        compiler_params=pltpu.CompilerParams(
            dimension_semantics=("arbitrary", "arbitrary")),
    )(tile_g, tile_b, tile_v, offs9, xs, wsort, W1, b1r, W2, b2r)


def _combine_body(pos_hbm, hs_hbm, out_hbm, pv0, pv1, bufa, bufb, sem):
    wid = lax.axis_index("s") * NC + lax.axis_index("c")
    ntok = T // NW
    half = ntok // 2
    for hp in range(2):
        tbase = wid * ntok + hp * half
        pltpu.sync_copy(pos_hbm.at[pl.ds(tbase, half)], pv0)
        pltpu.sync_copy(pos_hbm.at[pl.ds(T + tbase, half)], pv1)
        pltpu.async_copy(hs_hbm.at[pv0], bufa, sem).wait()
        pltpu.async_copy(hs_hbm.at[pv1], bufb, sem).wait()

        def _row(r, _):
            for c in range(D // 16):
                sl = pl.ds(c * 16, 16)
                bufa[r, sl] = bufa[r, sl] + bufb[r, sl]
            return 0

        lax.fori_loop(0, half, _row, 0)
        pltpu.sync_copy(bufa, out_hbm.at[pl.ds(tbase, half)])


_combine = functools.partial(
    pl.kernel,
    out_type=jax.ShapeDtypeStruct((T, D), jnp.float32),
    mesh=plsc.VectorSubcoreMesh(core_axis_name="c", subcore_axis_name="s",
                                num_cores=NC, num_subcores=NS),
    scratch_types=[
        pltpu.VMEM((T // NW // 2,), jnp.int32),
        pltpu.VMEM((T // NW // 2,), jnp.int32),
        pltpu.VMEM((T // NW // 2, D), jnp.float32),
        pltpu.VMEM((T // NW // 2, D), jnp.float32),
        pltpu.SemaphoreType.DMA,
    ],
)(_combine_body)


def _tile_metadata(offs9):
    counts = offs9[1:] - offs9[:8]
    sb = offs9[:8] // RB
    eb = (offs9[1:] + RB - 1) // RB
    ntiles = jnp.where(counts > 0, eb - sb, 0)
    tstart = jnp.concatenate([jnp.zeros((1,), jnp.int32),
                              jnp.cumsum(ntiles).astype(jnp.int32)])
    total = tstart[8]
    i = jnp.arange(MAXT, dtype=jnp.int32)
    g = jnp.searchsorted(tstart[1:], i, side="right").astype(jnp.int32)
    valid = (i < total).astype(jnp.int32)
    g = jnp.minimum(g, 7)
    b = jnp.clip(sb[g] + (i - tstart[g]), 0, NB - 1)
    return g, b, valid


@jax.jit
def kernel(x, Wr, br, W1, b1, W2, b2):
    logits = x @ Wr + br
    logits_p = jnp.pad(logits, ((0, 0), (0, 128 - E)),
                       constant_values=-1e30)
    pos2d, wrow, offs_row = _router(logits_p)
    pos = pos2d.reshape(P)
    offs9 = offs_row[0, :9]

    xs, wsort = _dispatch(pos, wrow, x)

    tile_g, tile_b, tile_v = _tile_metadata(offs9)
    hs = _gmm(tile_g, tile_b, tile_v, offs9, xs, wsort,
              W1, b1.reshape(E, 1, H), W2, b2.reshape(E, 1, D))

    return _combine(pos, hs)


# speedup vs baseline: 1.0172x; 1.0172x over previous
import functools

import jax
import jax.numpy as jnp
from jax import lax
from jax.experimental import pallas as pl
from jax.experimental.pallas import tpu as pltpu
from jax.experimental.pallas import tpu_sc as plsc

T = 2048
D = 1024
E = 8
H = 4 * D
P = 2 * T
CH = 256
RB = 256
NB = P // RB
MAXT = NB + E - 1
HBLK = 2048
NHB = H // HBLK

NC = 2
NS = 16
NW = NC * NS


def _router_body(lg_ref, pos_ref, wrow_ref, offs_ref):
    lane = lax.broadcasted_iota(jnp.int32, (T, 128), 1)
    neg = jnp.float32(-1e30)
    logits = jnp.where(lane < E, lg_ref[...], neg)
    m1 = jnp.max(logits, axis=1, keepdims=True)
    i1 = jnp.min(jnp.where(logits == m1, lane, 127), axis=1, keepdims=True)
    l2 = jnp.where(lane == i1, neg, logits)
    m2 = jnp.max(l2, axis=1, keepdims=True)
    i2 = jnp.min(jnp.where(l2 == m2, lane, 127), axis=1, keepdims=True)
    w1 = 1.0 / (1.0 + jnp.exp(m2 - m1))
    w2 = 1.0 - w1

    oh1 = (lane == i1).astype(jnp.float32)
    oh2 = (lane == i2).astype(jnp.float32)

    li = lax.broadcasted_iota(jnp.int32, (CH, CH), 0)
    lj = lax.broadcasted_iota(jnp.int32, (CH, CH), 1)
    ltri = (li > lj).astype(jnp.float32)

    off = jnp.zeros((1, 128), jnp.float32)
    ranks = []
    for c in range(2 * T // CH):
        src = oh1 if c < T // CH else oh2
        cc = c % (T // CH)
        chunk = src[cc * CH:(cc + 1) * CH, :]
        ranks.append(jnp.dot(ltri, chunk, precision=lax.Precision.HIGHEST,
                             preferred_element_type=jnp.float32) + off)
        off = off + jnp.sum(chunk, axis=0, keepdims=True)

    counts = off
    ui = lax.broadcasted_iota(jnp.int32, (128, 128), 0)
    uj = lax.broadcasted_iota(jnp.int32, (128, 128), 1)
    uppr = (ui < uj).astype(jnp.float32)
    c_hi = jnp.floor(counts * (1.0 / 64.0))
    c_lo = counts - 64.0 * c_hi
    offs = 64.0 * jnp.dot(c_hi, uppr, precision=lax.Precision.HIGHEST,
                          preferred_element_type=jnp.float32) \
        + jnp.dot(c_lo, uppr, precision=lax.Precision.HIGHEST,
                  preferred_element_type=jnp.float32)

    rank_all = jnp.concatenate(ranks, axis=0)
    oh_all = jnp.concatenate([oh1, oh2], axis=0)
    pos = jnp.sum(oh_all * (rank_all + offs), axis=1, keepdims=True)
    pos_ref[...] = pos.astype(jnp.int32)
    w_all = jnp.concatenate([w1, w2], axis=0)
    wrow_ref[...] = jnp.broadcast_to(w_all, (P, 128))
    offs_ref[...] = offs.astype(jnp.int32)


def _router(logits_p):
    return pl.pallas_call(
        _router_body,
        in_specs=[
            pl.BlockSpec((T, 128), lambda: (0, 0)),
        ],
        out_specs=[
            pl.BlockSpec((P, 1), lambda: (0, 0)),
            pl.BlockSpec((P, 128), lambda: (0, 0)),
            pl.BlockSpec((1, 128), lambda: (0, 0)),
        ],
        out_shape=[
            jax.ShapeDtypeStruct((P, 1), jnp.int32),
            jax.ShapeDtypeStruct((P, 128), jnp.float32),
            jax.ShapeDtypeStruct((1, 128), jnp.int32),
        ],
    )(logits_p)


def _dispatch_body(pos_hbm, wrow_hbm, x_hbm, xs_hbm, wsort_hbm,
                   posv, tokv, rowbuf, wbuf, sem):
    wid = lax.axis_index("s") * NC + lax.axis_index("c")
    npair = P // NW
    half = npair // 2
    for hp in range(2):
        base = wid * npair + hp * half
        pltpu.sync_copy(pos_hbm.at[pl.ds(base, half)], posv)
        for j in range(half // 16):
            t16 = (base + j * 16 + lax.iota(jnp.int32, 16)) & (T - 1)
            tokv[pl.ds(j * 16, 16)] = t16
        pltpu.async_copy(x_hbm.at[tokv], rowbuf, sem).wait()
        pltpu.sync_copy(wrow_hbm.at[pl.ds(base, half)], wbuf)
        pltpu.async_copy(rowbuf, xs_hbm.at[posv], sem).wait()
        pltpu.async_copy(wbuf, wsort_hbm.at[posv], sem).wait()


_dispatch = functools.partial(
    pl.kernel,
    out_type=[
        jax.ShapeDtypeStruct((P, D), jnp.float32),
        jax.ShapeDtypeStruct((P, 128), jnp.float32),
    ],
    mesh=plsc.VectorSubcoreMesh(core_axis_name="c", subcore_axis_name="s",
                                num_cores=NC, num_subcores=NS),
    scratch_types=[
        pltpu.VMEM((P // NW // 2,), jnp.int32),
        pltpu.VMEM((P // NW // 2,), jnp.int32),
        pltpu.VMEM((P // NW // 2, D), jnp.float32),
        pltpu.VMEM((P // NW // 2, 128), jnp.float32),
        pltpu.SemaphoreType.DMA,
    ],
)(_dispatch_body)


def _gmm_body(tg_ref, tb_ref, tv_ref, offs_ref,
              xs_ref, ws_ref, w1_ref, b1_ref, w2_ref, b2_ref, out_ref):
    h = pl.program_id(0)
    i = pl.program_id(1)

    @pl.when((h == 0) & (i == 0))
    def _init():
        out_ref[...] = jnp.zeros_like(out_ref)

    @pl.when(tv_ref[i] == 1)
    def _compute():
        g = tg_ref[i]
        b = tb_ref[i]
        r0 = b * RB
        riota = r0 + lax.broadcasted_iota(jnp.int32, (RB, 1), 0)
        active = (riota >= offs_ref[g]) & (riota < offs_ref[g + 1])

        hpre = jnp.dot(xs_ref[...], w1_ref[0],
                       preferred_element_type=jnp.float32) + b1_ref[0]
        hact = 0.5 * hpre * (1.0 + lax.erf(hpre * 0.7071067811865476))
        acc = jnp.dot(hact, w2_ref[0], preferred_element_type=jnp.float32)

        wcol = jnp.where(active, ws_ref[:, :1], 0.0)
        contrib = wcol * (acc + jnp.where(h == 0, 1.0, 0.0) * b2_ref[0])
        out_ref[pl.ds(r0, RB), :] += contrib


def _gmm(tile_g, tile_b, tile_v, offs9, xs, wsort, W1, b1r, W2, b2r):
    grid_spec = pltpu.PrefetchScalarGridSpec(
        num_scalar_prefetch=4,
        grid=(NHB, MAXT),
        in_specs=[
            pl.BlockSpec((RB, D), lambda h, i, tg, tb, tv, of: (tb[i], 0)),
            pl.BlockSpec((RB, 128), lambda h, i, tg, tb, tv, of: (tb[i], 0)),
            pl.BlockSpec((1, D, HBLK),
                         lambda h, i, tg, tb, tv, of: (tg[i], 0, h)),
            pl.BlockSpec((1, 1, HBLK),
                         lambda h, i, tg, tb, tv, of: (tg[i], 0, h)),
            pl.BlockSpec((1, HBLK, D),
                         lambda h, i, tg, tb, tv, of: (tg[i], h, 0)),
            pl.BlockSpec((1, 1, D),
                         lambda h, i, tg, tb, tv, of: (tg[i], 0, 0)),
        ],
        out_specs=pl.BlockSpec((P, D), lambda h, i, tg, tb, tv, of: (0, 0)),
    )
    return pl.pallas_call(
        _gmm_body,
        grid_spec=grid_spec,
        out_shape=jax.ShapeDtypeStruct((P, D), jnp.float32),
        compiler_params=pltpu.CompilerParams(
            dimension_semantics=("arbitrary", "arbitrary")),
    )(tile_g, tile_b, tile_v, offs9, xs, wsort, W1, b1r, W2, b2r)


def _combine_body(pos_hbm, hs_hbm, out_hbm, pv0, pv1, bufa, bufb, sem):
    wid = lax.axis_index("s") * NC + lax.axis_index("c")
    ntok = T // NW
    half = ntok // 2
    for hp in range(2):
        tbase = wid * ntok + hp * half
        pltpu.sync_copy(pos_hbm.at[pl.ds(tbase, half)], pv0)
        pltpu.sync_copy(pos_hbm.at[pl.ds(T + tbase, half)], pv1)
        pltpu.async_copy(hs_hbm.at[pv0], bufa, sem).wait()
        pltpu.async_copy(hs_hbm.at[pv1], bufb, sem).wait()

        def _row(r, _):
            for c in range(D // 16):
                sl = pl.ds(c * 16, 16)
                bufa[r, sl] = bufa[r, sl] + bufb[r, sl]
            return 0

        lax.fori_loop(0, half, _row, 0)
        pltpu.sync_copy(bufa, out_hbm.at[pl.ds(tbase, half)])


_combine = functools.partial(
    pl.kernel,
    out_type=jax.ShapeDtypeStruct((T, D), jnp.float32),
    mesh=plsc.VectorSubcoreMesh(core_axis_name="c", subcore_axis_name="s",
                                num_cores=NC, num_subcores=NS),
    scratch_types=[
        pltpu.VMEM((T // NW // 2,), jnp.int32),
        pltpu.VMEM((T // NW // 2,), jnp.int32),
        pltpu.VMEM((T // NW // 2, D), jnp.float32),
        pltpu.VMEM((T // NW // 2, D), jnp.float32),
        pltpu.SemaphoreType.DMA,
    ],
)(_combine_body)


def _tile_metadata(offs9):
    counts = offs9[1:] - offs9[:8]
    sb = offs9[:8] // RB
    eb = (offs9[1:] + RB - 1) // RB
    ntiles = jnp.where(counts > 0, eb - sb, 0)
    tstart = jnp.concatenate([jnp.zeros((1,), jnp.int32),
                              jnp.cumsum(ntiles).astype(jnp.int32)])
    total = tstart[8]
    i = jnp.arange(MAXT, dtype=jnp.int32)
    g = jnp.searchsorted(tstart[1:], i, side="right").astype(jnp.int32)
    valid = (i < total).astype(jnp.int32)
    g = jnp.minimum(g, 7)
    b = jnp.clip(sb[g] + (i - tstart[g]), 0, NB - 1)
    return g, b, valid


@jax.jit
def kernel(x, Wr, br, W1, b1, W2, b2):
    logits = x @ Wr + br
    logits_p = jnp.pad(logits, ((0, 0), (0, 128 - E)),
                       constant_values=-1e30)
    pos2d, wrow, offs_row = _router(logits_p)
    pos = pos2d.reshape(P)
    offs9 = offs_row[0, :9]

    xs, wsort = _dispatch(pos, wrow, x)

    tile_g, tile_b, tile_v = _tile_metadata(offs9)
    hs = _gmm(tile_g, tile_b, tile_v, offs9, xs, wsort,
              W1, b1.reshape(E, 1, H), W2, b2.reshape(E, 1, D))

    return _combine(pos, hs)
